# Initial kernel scaffold; baseline (speedup 1.0000x reference)
import jax
import jax.numpy as jnp
from jax.experimental import pallas as pl


def kernel(x, z, num_atoms, edges, emb, params):
    raise NotImplementedError("write your pallas kernel here")


# speedup vs baseline: 3.2708x; 3.2708x over previous
import functools

import jax
import jax.numpy as jnp
from jax import lax
from jax.experimental import pallas as pl
from jax.experimental.pallas import tpu as pltpu
from jax.experimental.pallas import tpu_sc as plsc

N = 10000
N_PAD = 10240
E = 160000
F = 128
TW = 2 * F
ZPAD = 128
NCORE = 2
NSUB = 16
NWORK = NCORE * NSUB
C = 128
NCH = E // C
GIT = -(-NCH // NWORK)
SIT = -(-NCH // NSUB)
RPT = N_PAD // NSUB
BE = 2000
BN = 1024

f32 = jnp.float32
i32 = jnp.int32


@functools.cache
def _sc_kernels():
    mesh = plsc.VectorSubcoreMesh(core_axis_name="c", subcore_axis_name="s",
                                  num_cores=NCORE, num_subcores=NSUB)

    @functools.partial(
        pl.kernel,
        out_type=(
            jax.ShapeDtypeStruct((E, TW), f32),
            jax.ShapeDtypeStruct((E, TW), f32),
        ),
        mesh=mesh,
        scratch_types=(
            pltpu.VMEM((C,), i32),
            pltpu.VMEM((C, TW), f32),
            pltpu.SemaphoreType.DMA,
        ),
    )
    def gather_k(tab, e0, e1, g0, g1, idx_v, row_v, sem):
        wid = lax.axis_index("s") * NCORE + lax.axis_index("c")

        def chunk(i, carry):
            ch = wid + i * NWORK

            @pl.when(ch < NCH)
            def _():
                off = ch * C
                for e, g in ((e0, g0), (e1, g1)):
                    pltpu.sync_copy(e.at[pl.ds(off, C)], idx_v)
                    pltpu.async_copy(tab.at[idx_v], row_v, sem).wait()
                    pltpu.sync_copy(row_v, g.at[pl.ds(off, C)])

            return carry

        lax.fori_loop(0, GIT, chunk, 0)

    @functools.partial(
        pl.kernel,
        out_type=(
            jax.ShapeDtypeStruct((N_PAD, F), f32),
            jax.ShapeDtypeStruct((N_PAD, F), f32),
        ),
        mesh=mesh,
        scratch_types=(
            pltpu.VMEM((C,), i32),
            pltpu.VMEM((C, F), f32),
            pltpu.VMEM_SHARED((N_PAD, F), f32),
        ),
    )
    def scatter_mx_k(m, xij, e0, zf, out_m, out_x, idx_v, row_v, sh):
        cid = lax.axis_index("c")
        sid = lax.axis_index("s")
        r0 = sid * RPT
        pltpu.sync_copy(zf, sh.at[pl.ds(r0, RPT)])
        plsc.subcore_barrier()

        def chunk_from(src):
            def chunk(i, carry):
                ch = sid + i * NSUB

                @pl.when(ch < NCH)
                def _():
                    off = ch * C
                    pltpu.sync_copy(e0.at[pl.ds(off, C)], idx_v)
                    pltpu.sync_copy(src.at[pl.ds(off, C)], row_v)
                    pltpu.sync_copy(row_v, sh.at[idx_v], add=True)

                return carry
            return chunk

        @pl.when(cid == 0)
        def _():
            lax.fori_loop(0, SIT, chunk_from(m), 0)

        @pl.when(cid == 1)
        def _():
            lax.fori_loop(0, SIT, chunk_from(xij), 0)

        plsc.subcore_barrier()

        @pl.when(cid == 0)
        def _():
            pltpu.sync_copy(sh.at[pl.ds(r0, RPT)], out_m.at[pl.ds(r0, RPT)])

        @pl.when(cid == 1)
        def _():
            pltpu.sync_copy(sh.at[pl.ds(r0, RPT)], out_x.at[pl.ds(r0, RPT)])

    @functools.partial(
        pl.kernel,
        out_type=jax.ShapeDtypeStruct((NCORE, N_PAD, F), f32),
        mesh=mesh,
        scratch_types=(
            pltpu.VMEM((C,), i32),
            pltpu.VMEM((C, F), f32),
            pltpu.VMEM_SHARED((N_PAD, F), f32),
        ),
    )
    def scatter_x_k(xij, e0, zf, out_x, idx_v, row_v, sh):
        cid = lax.axis_index("c")
        sid = lax.axis_index("s")
        wid = sid * NCORE + cid
        r0 = sid * RPT
        pltpu.sync_copy(zf, sh.at[pl.ds(r0, RPT)])
        plsc.subcore_barrier()

        def chunk(i, carry):
            ch = wid + i * NWORK

            @pl.when(ch < NCH)
            def _():
                off = ch * C
                pltpu.sync_copy(e0.at[pl.ds(off, C)], idx_v)
                pltpu.sync_copy(xij.at[pl.ds(off, C)], row_v)
                pltpu.sync_copy(row_v, sh.at[idx_v], add=True)

            return carry

        lax.fori_loop(0, GIT, chunk, 0)
        plsc.subcore_barrier()
        pltpu.sync_copy(sh.at[pl.ds(r0, RPT)], out_x.at[cid, pl.ds(r0, RPT)])

    return gather_k, scatter_mx_k, scatter_x_k


def _silu(v):
    return v * jax.nn.sigmoid(v)


def _mm(a, b):
    return jnp.dot(a, b, preferred_element_type=f32)


def _embed_body(z_ref, emb_ref, x_ref, out_ref):
    z = z_ref[...]
    col = lax.broadcasted_iota(i32, (BN, ZPAD), 1)
    oh = (col == z).astype(f32)
    h = _mm(oh, emb_ref[...])
    out_ref[...] = jnp.concatenate([h, x_ref[...]], axis=1)


def _edge_core(g0t, g1t, hij, a0, a1, a2, beu1, weu2, beu2,
               m0, m1, mp, md, bm1, wm2, bm2, wp1, bp1, wp2, bp2):
    g0 = g0t[:, :F]
    g1 = g1t[:, :F]
    pre = _mm(g0, a0[...]) + _mm(g1, a1[...]) + beu1[...]
    if hij is not None:
        pre = pre + _mm(hij, a2[...])
    t = _silu(pre)
    hpij = _mm(t, weu2[...]) + beu2[...]
    if hij is not None:
        hpij = hij + hpij
    diff = g1t[:, F:] - g0t[:, F:]
    d = jnp.sqrt(jnp.sum(diff * diff, axis=1, keepdims=True))
    u = _silu(_mm(g0, m0[...]) + _mm(g1, m1[...]) + _mm(hpij, mp[...])
              + d * md[...] + bm1[...])
    mm_ = _silu(_mm(u, wm2[...]) + bm2[...])
    w = _mm(_silu(_mm(mm_, wp1[...]) + bp1[...]), wp2[...]) + bp2[...]
    col = lax.broadcasted_iota(i32, (BE, F), 1)
    xij = jnp.where(col == 3, 1.0, jnp.where(col < 3, diff * w, 0.0))
    return hpij, mm_, xij


def _edge1_body(g0t, g1t, a0, a1, beu1, weu2, beu2,
                m0, m1, mp, md, bm1, wm2, bm2, wp1, bp1, wp2, bp2,
                hpij_o, m_o, xij_o):
    hpij, mm_, xij = _edge_core(g0t, g1t, None, a0, a1, None, beu1, weu2,
                                beu2, m0, m1, mp, md, bm1, wm2, bm2, wp1,
                                bp1, wp2, bp2)
    hpij_o[...] = hpij
    m_o[...] = mm_
    xij_o[...] = xij


def _edge2_body(g0t, g1t, hij_r, a0, a1, a2, beu1, weu2, beu2,
                m0, m1, mp, md, bm1, wm2, bm2, wp1, bp1, wp2, bp2, xij_o):
    _, _, xij = _edge_core(g0t, g1t, hij_r[...], a0, a1, a2, beu1, weu2,
                           beu2, m0, m1, mp, md, bm1, wm2, bm2, wp1, bp1,
                           wp2, bp2)
    xij_o[...] = xij


def _node_body(t_ref, sm_ref, sx_ref, n0, nm, bn1, wn2, bn2, out_ref):
    t = t_ref[...]
    h = t[:, :F]
    x = t[:, F:]
    sm = sm_ref[...]
    sx = sx_ref[...]
    col = lax.broadcasted_iota(i32, (BN, F), 1)
    cnt = jnp.sum(jnp.where(col == 3, sx, 0.0), axis=1, keepdims=True)
    recip = 1.0 / jnp.maximum(cnt, 1.0)
    m_i = sm * recip
    pre = _mm(h, n0[...]) + _mm(m_i, nm[...]) + bn1[...]
    hp = h + _mm(_silu(pre), wn2[...]) + bn2[...]
    xp = jnp.where(col < 3, x + sx * recip, 0.0)
    out_ref[...] = jnp.concatenate([hp, xp], axis=1)


def _final_body(t_ref, sx_ref, out_ref):
    x = t_ref[...][:, F:]
    sx3 = sx_ref[...]
    sx = sx3[0] + sx3[1]
    col = lax.broadcasted_iota(i32, (BN, F), 1)
    cnt = jnp.sum(jnp.where(col == 3, sx, 0.0), axis=1, keepdims=True)
    recip = 1.0 / jnp.maximum(cnt, 1.0)
    out_ref[...] = jnp.where(col < 3, x + sx * recip, 0.0)


def _full(shape):
    return pl.BlockSpec(shape, lambda i: tuple(0 for _ in shape))


_ET = pl.BlockSpec((BE, TW), lambda i: (i, 0))
_EH = pl.BlockSpec((BE, F), lambda i: (i, 0))
_NT = pl.BlockSpec((BN, TW), lambda i: (i, 0))
_NH = pl.BlockSpec((BN, F), lambda i: (i, 0))
_SX2 = pl.BlockSpec((NCORE, BN, F), lambda i: (0, i, 0))
_W = _full((F, F))
_B = _full((1, F))

_embed_call = pl.pallas_call(
    _embed_body,
    grid=(N_PAD // BN,),
    in_specs=[pl.BlockSpec((BN, 1), lambda i: (i, 0)), _full((ZPAD, F)), _NH],
    out_specs=_NT,
    out_shape=jax.ShapeDtypeStruct((N_PAD, TW), f32),
)

_edge1_call = pl.pallas_call(
    _edge1_body,
    grid=(E // BE,),
    in_specs=[_ET, _ET,
              _W, _W, _B, _W, _B,
              _W, _W, _W, _B, _B, _W, _B, _W, _B, _full((F, 1)), _full((1, 1))],
    out_specs=[_EH, _EH, _EH],
    out_shape=[jax.ShapeDtypeStruct((E, F), f32),
               jax.ShapeDtypeStruct((E, F), f32),
               jax.ShapeDtypeStruct((E, F), f32)],
)

_edge2_call = pl.pallas_call(
    _edge2_body,
    grid=(E // BE,),
    in_specs=[_ET, _ET, _EH,
              _W, _W, _W, _B, _W, _B,
              _W, _W, _W, _B, _B, _W, _B, _W, _B, _full((F, 1)), _full((1, 1))],
    out_specs=_EH,
    out_shape=jax.ShapeDtypeStruct((E, F), f32),
)

_node_call = pl.pallas_call(
    _node_body,
    grid=(N_PAD // BN,),
    in_specs=[_NT, _NH, _NH, _W, _W, _B, _W, _B],
    out_specs=_NT,
    out_shape=jax.ShapeDtypeStruct((N_PAD, TW), f32),
)

_final_call = pl.pallas_call(
    _final_body,
    grid=(N_PAD // BN,),
    in_specs=[_NT, _SX2],
    out_specs=_NH,
    out_shape=jax.ShapeDtypeStruct((N_PAD, F), f32),
)


def _row(b):
    return b.reshape(1, F)


def _edge_weights(p, with_a2):
    weu1, beu1 = p['eu1']
    weu2, beu2 = p['eu2']
    wm1, bm1 = p['msg1']
    wm2, bm2 = p['msg2']
    wp1, bp1 = p['pos1']
    wp2, bp2 = p['pos2']
    ws = [weu1[:F], weu1[F:2 * F]]
    if with_a2:
        ws.append(weu1[2 * F:])
    ws += [_row(beu1), weu2, _row(beu2),
           wm1[:F], wm1[F:2 * F], wm1[2 * F:3 * F], wm1[3 * F:3 * F + 1],
           _row(bm1), wm2, _row(bm2), wp1, _row(bp1), wp2,
           bp2.reshape(1, 1)]
    return ws


def _node_weights(p):
    wn1, bn1 = p['node1']
    wn2, bn2 = p['node2']
    return [wn1[:F], wn1[F:], _row(bn1), wn2, _row(bn2)]


def kernel(x, z, num_atoms, edges, emb, params):
    del num_atoms
    x = x.astype(f32)
    e0 = edges[0].astype(i32)
    e1 = edges[1].astype(i32)
    xp = jnp.pad(x, ((0, N_PAD - N), (0, F - 3)))
    zp = jnp.pad(z.astype(i32), (0, N_PAD - N)).reshape(N_PAD, 1)
    embp = jnp.pad(emb.astype(f32), ((0, ZPAD - emb.shape[0]), (0, 0)))
    zf = jnp.zeros((RPT, F), f32)
    _gather_k, _scatter_mx_k, _scatter_x_k = _sc_kernels()

    t1 = _embed_call(zp, embp, xp)
    g0, g1 = _gather_k(t1, e0, e1)
    hpij, m1, xij1 = _edge1_call(g0, g1, *_edge_weights(params[0], False))
    sm1, sx1 = _scatter_mx_k(m1, xij1, e0, zf)
    t2 = _node_call(t1, sm1, sx1, *_node_weights(params[0]))
    g0b, g1b = _gather_k(t2, e0, e1)
    xij2 = _edge2_call(g0b, g1b, hpij, *_edge_weights(params[1], True))
    sx2 = _scatter_x_k(xij2, e0, zf)
    xo = _final_call(t2, sx2)
    return xo[:N, :3]


# speedup vs baseline: 3.8107x; 1.1651x over previous
import functools

import jax
import jax.numpy as jnp
from jax import lax
from jax.experimental import pallas as pl
from jax.experimental.pallas import tpu as pltpu
from jax.experimental.pallas import tpu_sc as plsc

N = 10000
N_PAD = 10240
E = 160000
F = 128
XA = 16
ZPAD = 128
NCORE = 2
NSUB = 16
NWORK = NCORE * NSUB
C = 128
NCH = E // C
GIT = -(-NCH // NWORK)
SIT = -(-NCH // NSUB)
RPT = N_PAD // NSUB
BE = 2000
BN = 1024

f32 = jnp.float32
i32 = jnp.int32


@functools.cache
def _sc_kernels():
    mesh = plsc.VectorSubcoreMesh(core_axis_name="c", subcore_axis_name="s",
                                  num_cores=NCORE, num_subcores=NSUB)
    no_layout = pltpu.CompilerParams(needs_layout_passes=False)

    @functools.partial(
        pl.kernel,
        out_type=(
            jax.ShapeDtypeStruct((E, F), f32),
            jax.ShapeDtypeStruct((E, F), f32),
            jax.ShapeDtypeStruct((E, XA), f32),
        ),
        mesh=mesh,
        scratch_types=(
            pltpu.VMEM((C,), i32),
            pltpu.VMEM((C,), i32),
            pltpu.VMEM((C, F), f32),
            pltpu.VMEM((C, F), f32),
            pltpu.VMEM((C, XA), f32),
            pltpu.VMEM((N_PAD * 4,), f32),
            pltpu.SemaphoreType.DMA,
            pltpu.SemaphoreType.DMA,
        ),
        compiler_params=no_layout,
    )
    def gather_k(tab, xflat, e0, e1, g0, g1, aux, idx0_v, idx1_v, row0_v,
                 row1_v, aux_v, xtab_v, sem0, sem1):
        wid = lax.axis_index("s") * NCORE + lax.axis_index("c")
        pltpu.sync_copy(xflat, xtab_v)

        def chunk(i, carry):
            ch = wid + i * NWORK

            @pl.when(ch < NCH)
            def _():
                off = ch * C
                pltpu.sync_copy(e0.at[pl.ds(off, C)], idx0_v)
                pltpu.sync_copy(e1.at[pl.ds(off, C)], idx1_v)
                cp0 = pltpu.async_copy(tab.at[idx0_v], row0_v, sem0)
                cp1 = pltpu.async_copy(tab.at[idx1_v], row1_v, sem1)
                lanes = lax.iota(i32, 16)
                for g in range(C // 16):
                    i0 = idx0_v[pl.ds(g * 16, 16)] * 4
                    i1 = idx1_v[pl.ds(g * 16, 16)] * 4
                    rows = lanes + (g * 16)
                    dsq = jnp.zeros((16,), f32)
                    for c in range(3):
                        x0c = plsc.load_gather(xtab_v, [i0 + c])
                        x1c = plsc.load_gather(xtab_v, [i1 + c])
                        dc = x1c - x0c
                        dsq = dsq + dc * dc
                        plsc.store_scatter(aux_v, [rows, jnp.full((16,), c, i32)], dc)
                    plsc.store_scatter(aux_v, [rows, jnp.full((16,), 3, i32)], dsq)
                cp0.wait()
                cp1.wait()
                pltpu.sync_copy(row0_v, g0.at[pl.ds(off, C)])
                pltpu.sync_copy(row1_v, g1.at[pl.ds(off, C)])
                pltpu.sync_copy(aux_v, aux.at[pl.ds(off, C)])

            return carry

        lax.fori_loop(0, GIT, chunk, 0)

    @functools.partial(
        pl.kernel,
        out_type=(
            jax.ShapeDtypeStruct((N_PAD, F), f32),
            jax.ShapeDtypeStruct((N_PAD, F), f32),
        ),
        mesh=mesh,
        scratch_types=(
            pltpu.VMEM((C,), i32),
            pltpu.VMEM((C, F), f32),
            pltpu.VMEM_SHARED((N_PAD, F), f32),
        ),
    )
    def scatter_mx_k(m, xij, e0, zf, out_m, out_x, idx_v, row_v, sh):
        cid = lax.axis_index("c")
        sid = lax.axis_index("s")
        r0 = sid * RPT
        pltpu.sync_copy(zf, sh.at[pl.ds(r0, RPT)])
        plsc.subcore_barrier()

        def chunk_from(src):
            def chunk(i, carry):
                ch = sid + i * NSUB

                @pl.when(ch < NCH)
                def _():
                    off = ch * C
                    pltpu.sync_copy(e0.at[pl.ds(off, C)], idx_v)
                    pltpu.sync_copy(src.at[pl.ds(off, C)], row_v)
                    pltpu.sync_copy(row_v, sh.at[idx_v], add=True)

                return carry
            return chunk

        @pl.when(cid == 0)
        def _():
            lax.fori_loop(0, SIT, chunk_from(m), 0)

        @pl.when(cid == 1)
        def _():
            lax.fori_loop(0, SIT, chunk_from(xij), 0)

        plsc.subcore_barrier()

        @pl.when(cid == 0)
        def _():
            pltpu.sync_copy(sh.at[pl.ds(r0, RPT)], out_m.at[pl.ds(r0, RPT)])

        @pl.when(cid == 1)
        def _():
            pltpu.sync_copy(sh.at[pl.ds(r0, RPT)], out_x.at[pl.ds(r0, RPT)])

    @functools.partial(
        pl.kernel,
        out_type=jax.ShapeDtypeStruct((NCORE, N_PAD, F), f32),
        mesh=mesh,
        scratch_types=(
            pltpu.VMEM((C,), i32),
            pltpu.VMEM((C, F), f32),
            pltpu.VMEM_SHARED((N_PAD, F), f32),
        ),
    )
    def scatter_x_k(xij, e0, zf, out_x, idx_v, row_v, sh):
        cid = lax.axis_index("c")
        sid = lax.axis_index("s")
        wid = sid * NCORE + cid
        r0 = sid * RPT
        pltpu.sync_copy(zf, sh.at[pl.ds(r0, RPT)])
        plsc.subcore_barrier()

        def chunk(i, carry):
            ch = wid + i * NWORK

            @pl.when(ch < NCH)
            def _():
                off = ch * C
                pltpu.sync_copy(e0.at[pl.ds(off, C)], idx_v)
                pltpu.sync_copy(xij.at[pl.ds(off, C)], row_v)
                pltpu.sync_copy(row_v, sh.at[idx_v], add=True)

            return carry

        lax.fori_loop(0, GIT, chunk, 0)
        plsc.subcore_barrier()
        pltpu.sync_copy(sh.at[pl.ds(r0, RPT)], out_x.at[cid, pl.ds(r0, RPT)])

    return gather_k, scatter_mx_k, scatter_x_k


def _silu(v):
    return v * jax.nn.sigmoid(v)


def _mm(a, b):
    return jnp.dot(a, b, preferred_element_type=f32)


def _embed_body(z_ref, emb_ref, out_ref):
    z = z_ref[...]
    col = lax.broadcasted_iota(i32, (BN, ZPAD), 1)
    oh = (col == z).astype(f32)
    out_ref[...] = _mm(oh, emb_ref[...])


def _edge_core(g0h, g1h, aux_r, hij, a0, a1, a2, beu1, weu2, beu2,
               m0, m1, mp, md, bm1, wm2, bm2, wp1, bp1, wp2, bp2):
    g0 = g0h[...]
    g1 = g1h[...]
    pre = _mm(g0, a0[...]) + _mm(g1, a1[...]) + beu1[...]
    if hij is not None:
        pre = pre + _mm(hij, a2[...])
    t = _silu(pre)
    hpij = _mm(t, weu2[...]) + beu2[...]
    if hij is not None:
        hpij = hij + hpij
    aux = aux_r[...]
    col16 = lax.broadcasted_iota(i32, (BE, XA), 1)
    d = jnp.sqrt(jnp.sum(jnp.where(col16 == 3, aux, 0.0), axis=1,
                         keepdims=True))
    diffp = jnp.pad(jnp.where(col16 < 3, aux, 0.0), ((0, 0), (0, F - XA)))
    u = _silu(_mm(g0, m0[...]) + _mm(g1, m1[...]) + _mm(hpij, mp[...])
              + d * md[...] + bm1[...])
    mm_ = _silu(_mm(u, wm2[...]) + bm2[...])
    w = _mm(_silu(_mm(mm_, wp1[...]) + bp1[...]), wp2[...]) + bp2[...]
    col = lax.broadcasted_iota(i32, (BE, F), 1)
    xij = jnp.where(col == 3, 1.0, diffp * w)
    return hpij, mm_, xij


def _edge1_body(g0h, g1h, aux_r, a0, a1, beu1, weu2, beu2,
                m0, m1, mp, md, bm1, wm2, bm2, wp1, bp1, wp2, bp2,
                hpij_o, m_o, xij_o):
    hpij, mm_, xij = _edge_core(g0h, g1h, aux_r, None, a0, a1, None, beu1,
                                weu2, beu2, m0, m1, mp, md, bm1, wm2, bm2,
                                wp1, bp1, wp2, bp2)
    hpij_o[...] = hpij
    m_o[...] = mm_
    xij_o[...] = xij


def _edge2_body(g0h, g1h, aux_r, hij_r, a0, a1, a2, beu1, weu2, beu2,
                m0, m1, mp, md, bm1, wm2, bm2, wp1, bp1, wp2, bp2, xij_o):
    _, _, xij = _edge_core(g0h, g1h, aux_r, hij_r[...], a0, a1, a2, beu1,
                           weu2, beu2, m0, m1, mp, md, bm1, wm2, bm2, wp1,
                           bp1, wp2, bp2)
    xij_o[...] = xij


def _node_body(h_ref, x_ref, sm_ref, sx_ref, n0, nm, bn1, wn2, bn2,
               h_out, x_out):
    h = h_ref[...]
    x = x_ref[...]
    sm = sm_ref[...]
    sx = sx_ref[...]
    col = lax.broadcasted_iota(i32, (BN, F), 1)
    cnt = jnp.sum(jnp.where(col == 3, sx, 0.0), axis=1, keepdims=True)
    recip = 1.0 / jnp.maximum(cnt, 1.0)
    m_i = sm * recip
    pre = _mm(h, n0[...]) + _mm(m_i, nm[...]) + bn1[...]
    h_out[...] = h + _mm(_silu(pre), wn2[...]) + bn2[...]
    x_out[...] = jnp.where(col < 3, x + sx * recip, 0.0)


def _final_body(x_ref, sx_ref, out_ref):
    x = x_ref[...]
    sx3 = sx_ref[...]
    sx = sx3[0] + sx3[1]
    col = lax.broadcasted_iota(i32, (BN, F), 1)
    cnt = jnp.sum(jnp.where(col == 3, sx, 0.0), axis=1, keepdims=True)
    recip = 1.0 / jnp.maximum(cnt, 1.0)
    out_ref[...] = jnp.where(col < 3, x + sx * recip, 0.0)


def _full(shape):
    return pl.BlockSpec(shape, lambda i: tuple(0 for _ in shape))


_EH = pl.BlockSpec((BE, F), lambda i: (i, 0))
_EA = pl.BlockSpec((BE, XA), lambda i: (i, 0))
_NH = pl.BlockSpec((BN, F), lambda i: (i, 0))
_SX2 = pl.BlockSpec((NCORE, BN, F), lambda i: (0, i, 0))
_W = _full((F, F))
_B = _full((1, F))

_embed_call = pl.pallas_call(
    _embed_body,
    grid=(N_PAD // BN,),
    in_specs=[pl.BlockSpec((BN, 1), lambda i: (i, 0)), _full((ZPAD, F))],
    out_specs=_NH,
    out_shape=jax.ShapeDtypeStruct((N_PAD, F), f32),
)

_edge1_call = pl.pallas_call(
    _edge1_body,
    grid=(E // BE,),
    in_specs=[_EH, _EH, _EA,
              _W, _W, _B, _W, _B,
              _W, _W, _W, _B, _B, _W, _B, _W, _B, _full((F, 1)), _full((1, 1))],
    out_specs=[_EH, _EH, _EH],
    out_shape=[jax.ShapeDtypeStruct((E, F), f32),
               jax.ShapeDtypeStruct((E, F), f32),
               jax.ShapeDtypeStruct((E, F), f32)],
)

_edge2_call = pl.pallas_call(
    _edge2_body,
    grid=(E // BE,),
    in_specs=[_EH, _EH, _EA, _EH,
              _W, _W, _W, _B, _W, _B,
              _W, _W, _W, _B, _B, _W, _B, _W, _B, _full((F, 1)), _full((1, 1))],
    out_specs=_EH,
    out_shape=jax.ShapeDtypeStruct((E, F), f32),
)

_node_call = pl.pallas_call(
    _node_body,
    grid=(N_PAD // BN,),
    in_specs=[_NH, _NH, _NH, _NH, _W, _W, _B, _W, _B],
    out_specs=[_NH, _NH],
    out_shape=[jax.ShapeDtypeStruct((N_PAD, F), f32),
               jax.ShapeDtypeStruct((N_PAD, F), f32)],
)

_final_call = pl.pallas_call(
    _final_body,
    grid=(N_PAD // BN,),
    in_specs=[_NH, _SX2],
    out_specs=_NH,
    out_shape=jax.ShapeDtypeStruct((N_PAD, F), f32),
)


def _row(b):
    return b.reshape(1, F)


def _edge_weights(p, with_a2):
    weu1, beu1 = p['eu1']
    weu2, beu2 = p['eu2']
    wm1, bm1 = p['msg1']
    wm2, bm2 = p['msg2']
    wp1, bp1 = p['pos1']
    wp2, bp2 = p['pos2']
    ws = [weu1[:F], weu1[F:2 * F]]
    if with_a2:
        ws.append(weu1[2 * F:])
    ws += [_row(beu1), weu2, _row(beu2),
           wm1[:F], wm1[F:2 * F], wm1[2 * F:3 * F], wm1[3 * F:3 * F + 1],
           _row(bm1), wm2, _row(bm2), wp1, _row(bp1), wp2,
           bp2.reshape(1, 1)]
    return ws


def _node_weights(p):
    wn1, bn1 = p['node1']
    wn2, bn2 = p['node2']
    return [wn1[:F], wn1[F:], _row(bn1), wn2, _row(bn2)]


def kernel(x, z, num_atoms, edges, emb, params):
    del num_atoms
    x = x.astype(f32)
    e0 = edges[0].astype(i32)
    e1 = edges[1].astype(i32)
    xp1 = jnp.pad(x, ((0, N_PAD - N), (0, F - 3)))
    xflat1 = jnp.pad(x, ((0, N_PAD - N), (0, 1))).reshape(-1)
    zp = jnp.pad(z.astype(i32), (0, N_PAD - N)).reshape(N_PAD, 1)
    embp = jnp.pad(emb.astype(f32), ((0, ZPAD - emb.shape[0]), (0, 0)))
    zf = jnp.zeros((RPT, F), f32)
    _gather_k, _scatter_mx_k, _scatter_x_k = _sc_kernels()

    h1 = _embed_call(zp, embp)
    g0, g1, aux1 = _gather_k(h1, xflat1, e0, e1)
    hpij, m1, xij1 = _edge1_call(g0, g1, aux1,
                                 *_edge_weights(params[0], False))
    sm1, sx1 = _scatter_mx_k(m1, xij1, e0, zf)
    h2, xp2 = _node_call(h1, xp1, sm1, sx1, *_node_weights(params[0]))
    xflat2 = xp2[:, :4].reshape(-1)
    g0b, g1b, aux2 = _gather_k(h2, xflat2, e0, e1)
    xij2 = _edge2_call(g0b, g1b, aux2, hpij, *_edge_weights(params[1], True))
    sx2 = _scatter_x_k(xij2, e0, zf)
    xo = _final_call(xp2, sx2)
    return xo[:N, :3]


# speedup vs baseline: 4.0813x; 1.0710x over previous
import functools

import jax
import jax.numpy as jnp
from jax import lax
from jax.experimental import pallas as pl
from jax.experimental.pallas import tpu as pltpu
from jax.experimental.pallas import tpu_sc as plsc

N = 10000
N_PAD = 10240
E = 160000
F = 128
XA = 16
ZPAD = 128
NCORE = 2
NSUB = 16
NWORK = NCORE * NSUB
C = 128
NCH = E // C
CG = 64
NCHG = E // CG
GIT = -(-NCHG // NWORK)
SIT = -(-NCH // NSUB)
XIT = -(-NCH // NWORK)
RPT = N_PAD // NSUB
BE = 2000
BN = 1024

f32 = jnp.float32
i32 = jnp.int32


@functools.cache
def _sc_kernels():
    mesh = plsc.VectorSubcoreMesh(core_axis_name="c", subcore_axis_name="s",
                                  num_cores=NCORE, num_subcores=NSUB)
    no_layout = pltpu.CompilerParams(needs_layout_passes=False)

    @functools.partial(
        pl.kernel,
        out_type=(
            jax.ShapeDtypeStruct((E, F), f32),
            jax.ShapeDtypeStruct((E, F), f32),
            jax.ShapeDtypeStruct((E, XA), f32),
        ),
        mesh=mesh,
        scratch_types=(
            pltpu.VMEM((2, CG), i32),
            pltpu.VMEM((2, CG), i32),
            pltpu.VMEM((2, CG, F), f32),
            pltpu.VMEM((2, CG, F), f32),
            pltpu.VMEM((2, CG, XA), f32),
            pltpu.VMEM((N_PAD * 4,), f32),
            pltpu.SemaphoreType.DMA,
            pltpu.SemaphoreType.DMA,
            pltpu.SemaphoreType.DMA,
            pltpu.SemaphoreType.DMA,
        ),
        compiler_params=no_layout,
    )
    def gather_k(tab, xflat, e0, e1, g0, g1, aux, idx0_v, idx1_v, row0_v,
                 row1_v, aux_v, xtab_v, sem0, sem1, semw0, semw1):
        wid = lax.axis_index("s") * NCORE + lax.axis_index("c")
        pltpu.sync_copy(xflat, xtab_v)
        semw = (semw0, semw1)

        def wait_wb(b, ch):
            off = ch * CG
            pltpu.make_async_copy(row0_v.at[b], g0.at[pl.ds(off, CG)], semw[b]).wait()
            pltpu.make_async_copy(row1_v.at[b], g1.at[pl.ds(off, CG)], semw[b]).wait()
            pltpu.make_async_copy(aux_v.at[b], aux.at[pl.ds(off, CG)], semw[b]).wait()

        def do_chunk(j, b):
            i = j * 2 + b
            ch = wid + i * NWORK

            @pl.when(ch < NCHG)
            def _():
                off = ch * CG
                pltpu.sync_copy(e0.at[pl.ds(off, CG)], idx0_v.at[b])
                pltpu.sync_copy(e1.at[pl.ds(off, CG)], idx1_v.at[b])

                @pl.when(j > 0)
                def _():
                    wait_wb(b, ch - 2 * NWORK)

                cp0 = pltpu.async_copy(tab.at[idx0_v.at[b]], row0_v.at[b], sem0)
                cp1 = pltpu.async_copy(tab.at[idx1_v.at[b]], row1_v.at[b], sem1)
                lanes = lax.iota(i32, 16)
                for g in range(CG // 16):
                    i0 = idx0_v[b, pl.ds(g * 16, 16)] * 4
                    i1 = idx1_v[b, pl.ds(g * 16, 16)] * 4
                    rows = lanes + (g * 16)
                    dsq = jnp.zeros((16,), f32)
                    for c in range(3):
                        x0c = plsc.load_gather(xtab_v, [i0 + c])
                        x1c = plsc.load_gather(xtab_v, [i1 + c])
                        dc = x1c - x0c
                        dsq = dsq + dc * dc
                        plsc.store_scatter(aux_v.at[b], [rows, jnp.full((16,), c, i32)], dc)
                    plsc.store_scatter(aux_v.at[b], [rows, jnp.full((16,), 3, i32)], dsq)
                cp0.wait()
                cp1.wait()
                pltpu.async_copy(row0_v.at[b], g0.at[pl.ds(off, CG)], semw[b])
                pltpu.async_copy(row1_v.at[b], g1.at[pl.ds(off, CG)], semw[b])
                pltpu.async_copy(aux_v.at[b], aux.at[pl.ds(off, CG)], semw[b])

        def pair(j, carry):
            do_chunk(j, 0)
            do_chunk(j, 1)
            return carry

        lax.fori_loop(0, (GIT + 1) // 2, pair, 0)
        nch_w = (NCHG - wid + NWORK - 1) // NWORK
        for b in range(2):
            last = nch_w - 1 - ((nch_w - 1 - b) % 2)
            wait_wb(b, wid + last * NWORK)

    @functools.partial(
        pl.kernel,
        out_type=(
            jax.ShapeDtypeStruct((N_PAD, F), f32),
            jax.ShapeDtypeStruct((N_PAD, F), f32),
        ),
        mesh=mesh,
        scratch_types=(
            pltpu.VMEM((2, C), i32),
            pltpu.VMEM((2, C, F), f32),
            pltpu.VMEM_SHARED((N_PAD, F), f32),
            pltpu.SemaphoreType.DMA,
            pltpu.SemaphoreType.DMA,
        ),
        compiler_params=no_layout,
    )
    def scatter_mx_k(m, xij, e0, zf, out_m, out_x, idx_v, row_v, sh,
                     sema0, sema1):
        cid = lax.axis_index("c")
        sid = lax.axis_index("s")
        r0 = sid * RPT
        pltpu.sync_copy(zf, sh.at[pl.ds(r0, RPT)])
        plsc.subcore_barrier()
        sema = (sema0, sema1)

        def run_from(src):
            def do_chunk(j, b):
                ch = sid + (2 * j + b) * NSUB

                @pl.when(ch < NCH)
                def _():
                    off = ch * C

                    @pl.when(j > 0)
                    def _():
                        pltpu.make_async_copy(
                            row_v.at[b], sh.at[idx_v.at[b]], sema[b]).wait()

                    pltpu.sync_copy(e0.at[pl.ds(off, C)], idx_v.at[b])
                    pltpu.sync_copy(src.at[pl.ds(off, C)], row_v.at[b])
                    pltpu.async_copy(row_v.at[b], sh.at[idx_v.at[b]],
                                     sema[b], add=True)

            def pair(j, carry):
                do_chunk(j, 0)
                do_chunk(j, 1)
                return carry

            lax.fori_loop(0, (SIT + 1) // 2, pair, 0)
            for b in range(2):
                pltpu.make_async_copy(row_v.at[b], sh.at[idx_v.at[b]],
                                      sema[b]).wait()

        @pl.when(cid == 0)
        def _():
            run_from(m)

        @pl.when(cid == 1)
        def _():
            run_from(xij)

        plsc.subcore_barrier()

        @pl.when(cid == 0)
        def _():
            pltpu.sync_copy(sh.at[pl.ds(r0, RPT)], out_m.at[pl.ds(r0, RPT)])

        @pl.when(cid == 1)
        def _():
            pltpu.sync_copy(sh.at[pl.ds(r0, RPT)], out_x.at[pl.ds(r0, RPT)])

    @functools.partial(
        pl.kernel,
        out_type=jax.ShapeDtypeStruct((NCORE, N_PAD, F), f32),
        mesh=mesh,
        scratch_types=(
            pltpu.VMEM((2, C), i32),
            pltpu.VMEM((2, C, F), f32),
            pltpu.VMEM_SHARED((N_PAD, F), f32),
            pltpu.SemaphoreType.DMA,
            pltpu.SemaphoreType.DMA,
        ),
        compiler_params=no_layout,
    )
    def scatter_x_k(xij, e0, zf, out_x, idx_v, row_v, sh, sema0, sema1):
        cid = lax.axis_index("c")
        sid = lax.axis_index("s")
        wid = sid * NCORE + cid
        r0 = sid * RPT
        pltpu.sync_copy(zf, sh.at[pl.ds(r0, RPT)])
        plsc.subcore_barrier()
        sema = (sema0, sema1)

        def do_chunk(j, b):
            ch = wid + (2 * j + b) * NWORK

            @pl.when(ch < NCH)
            def _():
                off = ch * C

                @pl.when(j > 0)
                def _():
                    pltpu.make_async_copy(
                        row_v.at[b], sh.at[idx_v.at[b]], sema[b]).wait()

                pltpu.sync_copy(e0.at[pl.ds(off, C)], idx_v.at[b])
                pltpu.sync_copy(xij.at[pl.ds(off, C)], row_v.at[b])
                pltpu.async_copy(row_v.at[b], sh.at[idx_v.at[b]],
                                 sema[b], add=True)

        def pair(j, carry):
            do_chunk(j, 0)
            do_chunk(j, 1)
            return carry

        lax.fori_loop(0, (XIT + 1) // 2, pair, 0)
        for b in range(2):
            pltpu.make_async_copy(row_v.at[b], sh.at[idx_v.at[b]],
                                  sema[b]).wait()
        plsc.subcore_barrier()
        pltpu.sync_copy(sh.at[pl.ds(r0, RPT)], out_x.at[cid, pl.ds(r0, RPT)])

    return gather_k, scatter_mx_k, scatter_x_k


def _silu(v):
    return v * jax.nn.sigmoid(v)


def _mm(a, b):
    return jnp.dot(a, b, preferred_element_type=f32)


def _embed_body(z_ref, emb_ref, out_ref):
    z = z_ref[...]
    col = lax.broadcasted_iota(i32, (BN, ZPAD), 1)
    oh = (col == z).astype(f32)
    out_ref[...] = _mm(oh, emb_ref[...])


def _edge_core(g0h, g1h, aux_r, hij, a0, a1, a2, beu1, weu2, beu2,
               m0, m1, mp, md, bm1, wm2, bm2, wp1, bp1, wp2, bp2):
    g0 = g0h[...]
    g1 = g1h[...]
    pre = _mm(g0, a0[...]) + _mm(g1, a1[...]) + beu1[...]
    if hij is not None:
        pre = pre + _mm(hij, a2[...])
    t = _silu(pre)
    hpij = _mm(t, weu2[...]) + beu2[...]
    if hij is not None:
        hpij = hij + hpij
    aux = aux_r[...]
    col16 = lax.broadcasted_iota(i32, (BE, XA), 1)
    d = jnp.sqrt(jnp.sum(jnp.where(col16 == 3, aux, 0.0), axis=1,
                         keepdims=True))
    diffp = jnp.pad(jnp.where(col16 < 3, aux, 0.0), ((0, 0), (0, F - XA)))
    u = _silu(_mm(g0, m0[...]) + _mm(g1, m1[...]) + _mm(hpij, mp[...])
              + d * md[...] + bm1[...])
    mm_ = _silu(_mm(u, wm2[...]) + bm2[...])
    w = _mm(_silu(_mm(mm_, wp1[...]) + bp1[...]), wp2[...]) + bp2[...]
    col = lax.broadcasted_iota(i32, (BE, F), 1)
    xij = jnp.where(col == 3, 1.0, diffp * w)
    return hpij, mm_, xij


def _edge1_body(g0h, g1h, aux_r, a0, a1, beu1, weu2, beu2,
                m0, m1, mp, md, bm1, wm2, bm2, wp1, bp1, wp2, bp2,
                hpij_o, m_o, xij_o):
    hpij, mm_, xij = _edge_core(g0h, g1h, aux_r, None, a0, a1, None, beu1,
                                weu2, beu2, m0, m1, mp, md, bm1, wm2, bm2,
                                wp1, bp1, wp2, bp2)
    hpij_o[...] = hpij
    m_o[...] = mm_
    xij_o[...] = xij


def _edge2_body(g0h, g1h, aux_r, hij_r, a0, a1, a2, beu1, weu2, beu2,
                m0, m1, mp, md, bm1, wm2, bm2, wp1, bp1, wp2, bp2, xij_o):
    _, _, xij = _edge_core(g0h, g1h, aux_r, hij_r[...], a0, a1, a2, beu1,
                           weu2, beu2, m0, m1, mp, md, bm1, wm2, bm2, wp1,
                           bp1, wp2, bp2)
    xij_o[...] = xij


def _node_body(h_ref, x_ref, sm_ref, sx_ref, n0, nm, bn1, wn2, bn2,
               h_out, x_out):
    h = h_ref[...]
    x = x_ref[...]
    sm = sm_ref[...]
    sx = sx_ref[...]
    col = lax.broadcasted_iota(i32, (BN, F), 1)
    cnt = jnp.sum(jnp.where(col == 3, sx, 0.0), axis=1, keepdims=True)
    recip = 1.0 / jnp.maximum(cnt, 1.0)
    m_i = sm * recip
    pre = _mm(h, n0[...]) + _mm(m_i, nm[...]) + bn1[...]
    h_out[...] = h + _mm(_silu(pre), wn2[...]) + bn2[...]
    x_out[...] = jnp.where(col < 3, x + sx * recip, 0.0)


def _final_body(x_ref, sx_ref, out_ref):
    x = x_ref[...]
    sx3 = sx_ref[...]
    sx = sx3[0] + sx3[1]
    col = lax.broadcasted_iota(i32, (BN, F), 1)
    cnt = jnp.sum(jnp.where(col == 3, sx, 0.0), axis=1, keepdims=True)
    recip = 1.0 / jnp.maximum(cnt, 1.0)
    out_ref[...] = jnp.where(col < 3, x + sx * recip, 0.0)


def _full(shape):
    return pl.BlockSpec(shape, lambda i: tuple(0 for _ in shape))


_EH = pl.BlockSpec((BE, F), lambda i: (i, 0))
_EA = pl.BlockSpec((BE, XA), lambda i: (i, 0))
_NH = pl.BlockSpec((BN, F), lambda i: (i, 0))
_SX2 = pl.BlockSpec((NCORE, BN, F), lambda i: (0, i, 0))
_W = _full((F, F))
_B = _full((1, F))

_embed_call = pl.pallas_call(
    _embed_body,
    grid=(N_PAD // BN,),
    in_specs=[pl.BlockSpec((BN, 1), lambda i: (i, 0)), _full((ZPAD, F))],
    out_specs=_NH,
    out_shape=jax.ShapeDtypeStruct((N_PAD, F), f32),
)

_edge1_call = pl.pallas_call(
    _edge1_body,
    grid=(E // BE,),
    in_specs=[_EH, _EH, _EA,
              _W, _W, _B, _W, _B,
              _W, _W, _W, _B, _B, _W, _B, _W, _B, _full((F, 1)), _full((1, 1))],
    out_specs=[_EH, _EH, _EH],
    out_shape=[jax.ShapeDtypeStruct((E, F), f32),
               jax.ShapeDtypeStruct((E, F), f32),
               jax.ShapeDtypeStruct((E, F), f32)],
)

_edge2_call = pl.pallas_call(
    _edge2_body,
    grid=(E // BE,),
    in_specs=[_EH, _EH, _EA, _EH,
              _W, _W, _W, _B, _W, _B,
              _W, _W, _W, _B, _B, _W, _B, _W, _B, _full((F, 1)), _full((1, 1))],
    out_specs=_EH,
    out_shape=jax.ShapeDtypeStruct((E, F), f32),
)

_node_call = pl.pallas_call(
    _node_body,
    grid=(N_PAD // BN,),
    in_specs=[_NH, _NH, _NH, _NH, _W, _W, _B, _W, _B],
    out_specs=[_NH, _NH],
    out_shape=[jax.ShapeDtypeStruct((N_PAD, F), f32),
               jax.ShapeDtypeStruct((N_PAD, F), f32)],
)

_final_call = pl.pallas_call(
    _final_body,
    grid=(N_PAD // BN,),
    in_specs=[_NH, _SX2],
    out_specs=_NH,
    out_shape=jax.ShapeDtypeStruct((N_PAD, F), f32),
)


def _row(b):
    return b.reshape(1, F)


def _edge_weights(p, with_a2):
    weu1, beu1 = p['eu1']
    weu2, beu2 = p['eu2']
    wm1, bm1 = p['msg1']
    wm2, bm2 = p['msg2']
    wp1, bp1 = p['pos1']
    wp2, bp2 = p['pos2']
    ws = [weu1[:F], weu1[F:2 * F]]
    if with_a2:
        ws.append(weu1[2 * F:])
    ws += [_row(beu1), weu2, _row(beu2),
           wm1[:F], wm1[F:2 * F], wm1[2 * F:3 * F], wm1[3 * F:3 * F + 1],
           _row(bm1), wm2, _row(bm2), wp1, _row(bp1), wp2,
           bp2.reshape(1, 1)]
    return ws


def _node_weights(p):
    wn1, bn1 = p['node1']
    wn2, bn2 = p['node2']
    return [wn1[:F], wn1[F:], _row(bn1), wn2, _row(bn2)]


def kernel(x, z, num_atoms, edges, emb, params):
    del num_atoms
    x = x.astype(f32)
    e0 = edges[0].astype(i32)
    e1 = edges[1].astype(i32)
    xp1 = jnp.pad(x, ((0, N_PAD - N), (0, F - 3)))
    xflat1 = jnp.pad(x, ((0, N_PAD - N), (0, 1))).reshape(-1)
    zp = jnp.pad(z.astype(i32), (0, N_PAD - N)).reshape(N_PAD, 1)
    embp = jnp.pad(emb.astype(f32), ((0, ZPAD - emb.shape[0]), (0, 0)))
    zf = jnp.zeros((RPT, F), f32)
    _gather_k, _scatter_mx_k, _scatter_x_k = _sc_kernels()

    h1 = _embed_call(zp, embp)
    g0, g1, aux1 = _gather_k(h1, xflat1, e0, e1)
    hpij, m1, xij1 = _edge1_call(g0, g1, aux1,
                                 *_edge_weights(params[0], False))
    sm1, sx1 = _scatter_mx_k(m1, xij1, e0, zf)
    h2, xp2 = _node_call(h1, xp1, sm1, sx1, *_node_weights(params[0]))
    xflat2 = xp2[:, :4].reshape(-1)
    g0b, g1b, aux2 = _gather_k(h2, xflat2, e0, e1)
    xij2 = _edge2_call(g0b, g1b, aux2, hpij, *_edge_weights(params[1], True))
    sx2 = _scatter_x_k(xij2, e0, zf)
    xo = _final_call(xp2, sx2)
    return xo[:N, :3]
